# baseline (speedup 1.0000x reference)
import functools

import jax
import jax.numpy as jnp
from jax import lax
from jax.experimental import pallas as pl
from jax.experimental.pallas import tpu as pltpu
from jax.experimental.pallas import tpu_sc as plsc

N = 10000
D = 128
C = 16
K = 5000
N2 = 10240
CH = 640
NV = CH // 16
NEG_INF = float("-inf")


def _tc1_body(x_ref, w_ref, b_ref, wp_ref, predt_ref, z2_ref):
    x = x_ref[:, :]
    w = w_ref[:, :]
    bt = b_ref[:, :]
    wp = wp_ref[:, :]
    predt = jax.lax.dot_general(
        w, x, (((1,), (1,)), ((), ())), preferred_element_type=jnp.float32
    ) + bt
    predt_ref[:, :] = predt
    z = jax.lax.dot_general(
        wp, predt, (((1,), (0,)), ((), ())),
        preferred_element_type=jnp.float32)
    z2_ref[:, :] = jnp.full((1, N2), NEG_INF, jnp.float32)
    z2_ref[0:1, 0:N] = z


def _sc_body(z_hbm, out_hbm, zv, kv, cntv, bsum, totb, outv, shared):
    s = lax.axis_index("s")
    c = lax.axis_index("c")
    iota = lax.iota(jnp.int32, 16)
    ones = jnp.ones((16,), jnp.int32)
    zeros = jnp.zeros((16,), jnp.int32)

    def _lanesum(v):
        for h in (8, 4, 2, 1):
            bsum[...] = v
            v = v + plsc.load_gather(bsum, [jnp.bitwise_xor(iota, h)])
        return v

    pltpu.sync_copy(z_hbm.at[pl.ds(s * CH, CH)], zv)

    def _mk(i, carry):
        zb = zv[pl.ds(i * 16, 16)]
        u = jax.lax.bitcast_convert_type(zb, jnp.uint32)
        sign = u >> jnp.uint32(31)
        flip = jnp.where(sign == jnp.uint32(1),
                         jnp.uint32(0xFFFFFFFF), jnp.uint32(0x80000000))
        kv[pl.ds(i * 16, 16)] = u ^ flip
        return carry

    lax.fori_loop(0, NV, _mk, jnp.int32(0))

    def _share_counts(slot, packed):
        cntv[...] = packed
        pltpu.sync_copy(cntv, shared.at[slot, s])
        plsc.subcore_barrier()
        pltpu.sync_copy(shared.at[slot], totb)
        tot = lax.fori_loop(0, 16, lambda t, a: a + totb[t], zeros)
        return tot

    def _key_round(r, p):
        sh = jnp.uint32(28) - jnp.uint32(4) * r.astype(jnp.uint32)

        def _cnt(i, accs):
            kb = kv[pl.ds(i * 16, 16)]
            return tuple(
                accs[d - 1] + jnp.where(
                    kb >= (p | (jnp.uint32(d) << sh)), ones, zeros)
                for d in range(1, 16))

        accs = lax.fori_loop(0, NV, _cnt, tuple([zeros] * 15))
        packed = zeros
        for d in range(1, 16):
            packed = jnp.where(iota == d, _lanesum(accs[d - 1]), packed)
        tot = _share_counts(r, packed)
        digit = _lanesum(jnp.where((tot >= K) & (iota >= 1), ones, zeros))
        return p | (digit.astype(jnp.uint32) << sh)

    kth = lax.fori_loop(0, 8, _key_round,
                        jnp.zeros((16,), jnp.uint32))

    def _above(i, acc):
        kb = kv[pl.ds(i * 16, 16)]
        return acc + jnp.where(kb > kth, ones, zeros)

    macc = lax.fori_loop(0, NV, _above, zeros)
    packed = jnp.where(iota == 1, _lanesum(macc), zeros)
    mtot = _share_counts(8, packed)
    m = _lanesum(jnp.where(iota == 1, mtot, zeros))
    need = K - m

    base = s * CH

    def _tie_round(r, q):
        sh = jnp.int32(12) - jnp.int32(4) * r

        def _cnt(i, accs):
            kb = kv[pl.ds(i * 16, 16)]
            tie = kb == kth
            idxv = base + i * 16 + iota
            return tuple(
                accs[d - 1] + jnp.where(
                    tie & (idxv < (q | (d << sh))), ones, zeros)
                for d in range(1, 16))

        accs = lax.fori_loop(0, NV, _cnt, tuple([zeros] * 15))
        packed = zeros
        for d in range(1, 16):
            packed = jnp.where(iota == d, _lanesum(accs[d - 1]), packed)
        tot = _share_counts(9 + r, packed)
        digit = _lanesum(jnp.where((tot < need) & (iota >= 1), ones, zeros))
        return q | (digit << sh)

    jstar = lax.fori_loop(0, 4, _tie_round, zeros)

    @pl.when((s == 0) & (c == 0))
    def _write():
        res = jnp.where(iota == 0,
                        jax.lax.bitcast_convert_type(kth, jnp.int32),
                        jnp.where(iota == 1, jstar, 0))
        outv[...] = res
        pltpu.sync_copy(outv, out_hbm)


def _tc2_body(z2_ref, predt_ref, sc_ref, wp_ref, xf_ref, zs_ref, ws_ref):
    kth = jax.lax.bitcast_convert_type(sc_ref[0], jnp.uint32)
    jstar = sc_ref[1]
    wp = wp_ref[:, :]

    RB, MB = 10, N2 // 10
    for j in range(RB):
        zs_ref[j:j + 1, :] = z2_ref[0:1, pl.ds(j * MB, MB)]
    zd = zs_ref[:, :]

    u = jax.lax.bitcast_convert_type(zd, jnp.uint32)
    sign = u >> jnp.uint32(31)
    flip = jnp.where(sign == jnp.uint32(1),
                     jnp.uint32(0xFFFFFFFF), jnp.uint32(0x80000000))
    key = u ^ flip
    idx = (jax.lax.broadcasted_iota(jnp.int32, (RB, MB), 0) * MB
           + jax.lax.broadcasted_iota(jnp.int32, (RB, MB), 1))
    sel = (key > kth) | ((key == kth) & (idx <= jstar))
    norm = jnp.sqrt(jnp.sum(wp * wp)) + 1e-16
    wgt = jnp.where(sel, jnp.tanh(zd / norm), 0.0)

    for j in range(RB):
        ws_ref[0:1, pl.ds(j * MB, MB)] = wgt[j:j + 1, :]
    wsv = jax.lax.slice(ws_ref[:, :], (0, 0), (1, N))
    acc = jax.lax.dot_general(
        wsv, predt_ref[:, :], (((1,), (1,)), ((), ())),
        preferred_element_type=jnp.float32)
    xf_ref[:, :] = acc * (1.0 / K)


@functools.partial(jax.jit, static_argnames=())
def kernel(x, edge_index, batch, W, b, w_pool):
    del edge_index, batch
    bt = b.reshape(C, 1)
    wp2 = w_pool.reshape(1, C)

    predt, z2 = pl.pallas_call(
        _tc1_body,
        out_shape=(
            jax.ShapeDtypeStruct((C, N), jnp.float32),
            jax.ShapeDtypeStruct((1, N2), jnp.float32),
        ),
    )(x, W, bt, wp2)

    mesh = plsc.VectorSubcoreMesh(core_axis_name="c", subcore_axis_name="s")
    sc_sel = pl.kernel(
        _sc_body,
        mesh=mesh,
        compiler_params=pltpu.CompilerParams(needs_layout_passes=False),
        out_type=jax.ShapeDtypeStruct((16,), jnp.int32),
        scratch_types=[
            pltpu.VMEM((CH,), jnp.float32),
            pltpu.VMEM((CH,), jnp.uint32),
            pltpu.VMEM((16,), jnp.int32),
            pltpu.VMEM((16,), jnp.int32),
            pltpu.VMEM((16, 16), jnp.int32),
            pltpu.VMEM((16,), jnp.int32),
            pltpu.VMEM_SHARED((13, 16, 16), jnp.int32),
        ],
    )
    scalars = sc_sel(z2.reshape(N2))

    x_final = pl.pallas_call(
        _tc2_body,
        in_specs=[
            pl.BlockSpec(memory_space=pltpu.MemorySpace.VMEM),
            pl.BlockSpec(memory_space=pltpu.MemorySpace.VMEM),
            pl.BlockSpec(memory_space=pltpu.MemorySpace.SMEM),
            pl.BlockSpec(memory_space=pltpu.MemorySpace.VMEM),
        ],
        out_shape=jax.ShapeDtypeStruct((1, C), jnp.float32),
        scratch_shapes=[
            pltpu.VMEM((10, N2 // 10), jnp.float32),
            pltpu.VMEM((1, N2), jnp.float32),
        ],
    )(z2, predt, scalars, wp2)

    return (x_final, predt.T)


# speedup vs baseline: 4.2600x; 4.2600x over previous
import functools

import jax
import jax.numpy as jnp
from jax.experimental import pallas as pl
from jax.experimental.pallas import tpu as pltpu

N = 10000
D = 128
C = 16
K = 5000
R = 10
M = N // R


def _body(x_ref, w_ref, b_ref, wp_ref, xf_ref, predt_ref, zr_ref, zs_ref,
          ws_ref):
    x = x_ref[:, :]
    w = w_ref[:, :]
    bt = b_ref[:, :]
    wp = wp_ref[:, :]

    predt = jax.lax.dot_general(
        w, x, (((1,), (1,)), ((), ())), preferred_element_type=jnp.float32
    ) + bt
    predt_ref[:, :] = predt

    z = jax.lax.dot_general(
        wp, predt, (((1,), (0,)), ((), ())),
        preferred_element_type=jnp.float32)
    zr_ref[:, :] = z

    for j in range(R):
        zs_ref[j:j + 1, :] = zr_ref[0:1, pl.ds(j * M, M)]
    zd = zs_ref[:, :]

    u = jax.lax.bitcast_convert_type(zd, jnp.uint32)
    sign = u >> jnp.uint32(31)
    flip = jnp.where(sign == jnp.uint32(1),
                     jnp.uint32(0xFFFFFFFF), jnp.uint32(0x80000000))
    key = u ^ flip

    def _cnt_ge(t):
        return jnp.sum((key >= t).astype(jnp.int32))

    kth = jnp.uint32(0)
    for sh in range(28, -1, -4):
        cnts = [_cnt_ge(kth | jnp.uint32(d << sh)) for d in range(1, 16)]
        digit = sum((c >= K).astype(jnp.uint32) for c in cnts)
        kth = kth | (digit << jnp.uint32(sh))

    above = key > kth
    m = jnp.sum(above.astype(jnp.int32))
    need = K - m

    tie = key == kth
    idx = (jax.lax.broadcasted_iota(jnp.int32, (R, M), 0) * M
           + jax.lax.broadcasted_iota(jnp.int32, (R, M), 1))

    def _cnt_lt(v):
        return jnp.sum((tie & (idx < v)).astype(jnp.int32))

    def _tie_radix():
        js = jnp.int32(0)
        for sh in range(12, -1, -4):
            cnts = [_cnt_lt(js | jnp.int32(d << sh)) for d in range(1, 16)]
            digit = sum((c < need).astype(jnp.int32) for c in cnts)
            js = js | (digit << sh)
        return js

    t_total = jnp.sum(tie.astype(jnp.int32))
    jstar = jax.lax.cond(need == t_total, lambda: jnp.int32(N - 1),
                         _tie_radix)

    sel = above | (tie & (idx <= jstar))
    norm = jnp.sqrt(jnp.sum(wp * wp)) + 1e-16
    wgt = jnp.where(sel, jnp.tanh(zd / norm), 0.0)

    for j in range(R):
        ws_ref[0:1, pl.ds(j * M, M)] = wgt[j:j + 1, :]

    acc = jax.lax.dot_general(
        ws_ref[:, :], predt, (((1,), (1,)), ((), ())),
        preferred_element_type=jnp.float32)
    xf_ref[:, :] = acc * (1.0 / K)


@functools.partial(jax.jit, static_argnames=())
def kernel(x, edge_index, batch, W, b, w_pool):
    del edge_index, batch
    bt = b.reshape(C, 1)
    wp2 = w_pool.reshape(1, C)
    x_final, predt = pl.pallas_call(
        _body,
        out_shape=(
            jax.ShapeDtypeStruct((1, C), jnp.float32),
            jax.ShapeDtypeStruct((C, N), jnp.float32),
        ),
        scratch_shapes=[
            pltpu.VMEM((1, N), jnp.float32),
            pltpu.VMEM((R, M), jnp.float32),
            pltpu.VMEM((1, N), jnp.float32),
        ],
    )(x, W, bt, wp2)
    return (x_final, predt.T)
